# Initial kernel scaffold; baseline (speedup 1.0000x reference)
import jax
import jax.numpy as jnp
from jax.experimental import pallas as pl


def kernel(x, table):
    raise NotImplementedError("write your pallas kernel here")


# speedup vs baseline: 1.8383x; 1.8383x over previous
import functools

import jax
import jax.numpy as jnp
from jax import lax
from jax.experimental import pallas as pl
from jax.experimental.pallas import tpu as pltpu
from jax.experimental.pallas import tpu_sc as plsc

NUM_EMB = 1000000
DIM = 64
B_TOTAL = 16384 * 50

NC = 2
NS = 16
NW = NC * NS

CHUNK = 128
B_PER_W = B_TOTAL // NW
NCHUNKS = B_PER_W // CHUNK


def _gather_kernel(table_hbm, idx_hbm, out_hbm, idx_v, rows0, rows1, sem0, sem1):
    wid = lax.axis_index("s") * NC + lax.axis_index("c")
    base = wid * B_PER_W

    pltpu.sync_copy(idx_hbm.at[wid], idx_v)

    bufs = (rows0, rows1)
    sems = (sem0, sem1)

    def gather(c, buf, sem):
        return pltpu.make_async_copy(table_hbm.at[idx_v.at[c]], buf, sem)

    gather(0, rows0, sem0).start()

    def step(i, carry):
        g = i * 2
        for b in range(2):
            c = g + b

            @pl.when(c + 1 < NCHUNKS)
            def _():
                gather(c + 1, bufs[1 - b], sems[1 - b]).start()

            gather(c, bufs[b], sems[b]).wait()
            pltpu.sync_copy(bufs[b], out_hbm.at[pl.ds(base + c * CHUNK, CHUNK)])
        return carry

    lax.fori_loop(0, NCHUNKS // 2, step, 0)


@functools.partial(
    pl.kernel,
    mesh=plsc.VectorSubcoreMesh(core_axis_name="c", subcore_axis_name="s"),
    out_type=jax.ShapeDtypeStruct((B_TOTAL, DIM), jnp.float32),
    scratch_types=[
        pltpu.VMEM((NCHUNKS, CHUNK), jnp.int32),
        pltpu.VMEM((CHUNK, DIM), jnp.float32),
        pltpu.VMEM((CHUNK, DIM), jnp.float32),
        pltpu.SemaphoreType.DMA,
        pltpu.SemaphoreType.DMA,
    ],
    compiler_params=pltpu.CompilerParams(use_tc_tiling_on_sc=False),
)
def _lookup(table_hbm, idx_hbm, out_hbm, idx_v, rows0, rows1, sem0, sem1):
    _gather_kernel(table_hbm, idx_hbm, out_hbm, idx_v, rows0, rows1, sem0, sem1)


def kernel(x, table):
    batch, hist = x.shape
    idx = jnp.reshape(x.astype(jnp.int32), (NW, NCHUNKS, CHUNK))
    out = _lookup(table, idx)
    return jnp.reshape(out, (batch, hist, DIM))


# speedup vs baseline: 1.8794x; 1.0224x over previous
import functools

import jax
import jax.numpy as jnp
from jax import lax
from jax.experimental import pallas as pl
from jax.experimental.pallas import tpu as pltpu
from jax.experimental.pallas import tpu_sc as plsc

NUM_EMB = 1000000
DIM = 64
B_TOTAL = 16384 * 50

NC = 2
NS = 16
NW = NC * NS

CHUNK = 128
B_PER_W = B_TOTAL // NW
NCHUNKS = B_PER_W // CHUNK


NBUF = 4
AHEAD = NBUF - 1


def _gather_kernel(table_hbm, idx_hbm, out_hbm, idx_v, bufs, gsems, ssems):
    wid = lax.axis_index("s") * NC + lax.axis_index("c")
    base = wid * B_PER_W

    pltpu.sync_copy(idx_hbm.at[wid], idx_v)

    def gather(c, b):
        return pltpu.make_async_copy(table_hbm.at[idx_v.at[c]], bufs[b], gsems[b])

    def store(c, b):
        return pltpu.make_async_copy(
            bufs[b], out_hbm.at[pl.ds(base + c * CHUNK, CHUNK)], ssems[b]
        )

    for c in range(AHEAD):
        gather(c, c).start()

    def step(i, carry):
        g = i * NBUF
        for b in range(NBUF):
            c = g + b
            gather(c, b).wait()
            store(c, b).start()

            pb = (b - 1) % NBUF

            @pl.when(c >= 1)
            def _():
                store(c - 1, pb).wait()

            @pl.when(c + AHEAD < NCHUNKS)
            def _():
                gather(c + AHEAD, pb).start()
        return carry

    lax.fori_loop(0, NCHUNKS // NBUF, step, 0)
    store(NCHUNKS - 1, (NCHUNKS - 1) % NBUF).wait()


@functools.partial(
    pl.kernel,
    mesh=plsc.VectorSubcoreMesh(core_axis_name="c", subcore_axis_name="s"),
    out_type=jax.ShapeDtypeStruct((B_TOTAL, DIM), jnp.float32),
    scratch_types=[
        pltpu.VMEM((NCHUNKS, CHUNK), jnp.int32),
        [pltpu.VMEM((CHUNK, DIM), jnp.float32)] * NBUF,
        [pltpu.SemaphoreType.DMA] * NBUF,
        [pltpu.SemaphoreType.DMA] * NBUF,
    ],
    compiler_params=pltpu.CompilerParams(use_tc_tiling_on_sc=False),
)
def _lookup(table_hbm, idx_hbm, out_hbm, idx_v, bufs, gsems, ssems):
    _gather_kernel(table_hbm, idx_hbm, out_hbm, idx_v, bufs, gsems, ssems)


def kernel(x, table):
    batch, hist = x.shape
    idx = jnp.reshape(x.astype(jnp.int32), (NW, NCHUNKS, CHUNK))
    out = _lookup(table, idx)
    return jnp.reshape(out, (batch, hist, DIM))
